# baseline (device time: 265381 ns/iter reference)
import jax
import jax.numpy as jnp
from jax import lax
from jax.experimental import pallas as pl
from jax.experimental.pallas import tpu as pltpu

N_DEV = 32
B, SQ, SKV, DM = 2, 512, 512, 768
H_PER = 8
DH = 64
ROWS = B * SQ
CHUNK = ROWS // N_DEV

_MESH = pl.DeviceIdType.MESH


def _body(x_ref, wq_ref, k_hbm, v_hbm, wo_ref, out_ref,
          acc_ref, gather_ref, res_ref, k_vmem, v_vmem,
          send_a, recv_a, send_b, recv_b, local_sems):
    me = lax.axis_index("i")

    kv_copies = []
    for src, dst, i in ((k_hbm, k_vmem, 0), (v_hbm, v_vmem, 1)):
        c = pltpu.make_async_copy(
            src.at[:, :, pl.ds(me * H_PER, H_PER), :], dst, local_sems.at[i])
        c.start()
        kv_copies.append(c)

    barrier = pltpu.get_barrier_semaphore()

    def _sig(o, c):
        pl.semaphore_signal(barrier, inc=1,
                            device_id=(jnp.mod(me + o, N_DEV),),
                            device_id_type=_MESH)
        return c

    lax.fori_loop(1, N_DEV, _sig, 0)

    qi = lax.broadcasted_iota(jnp.int32, (SQ, SKV), 0)
    ki = lax.broadcasted_iota(jnp.int32, (SQ, SKV), 1)
    mask = (jnp.abs(qi - ki) <= 128) | (ki < 32) | (qi < 32)

    def _send_chunks(lo, hi):
        def f(t, c):
            @pl.when(t != me)
            def _():
                rdma = pltpu.make_async_remote_copy(
                    src_ref=acc_ref.at[pl.ds(t * CHUNK, CHUNK), :],
                    dst_ref=gather_ref.at[me],
                    send_sem=send_a.at[t],
                    recv_sem=recv_a.at[me],
                    device_id=(t,), device_id_type=_MESH,
                )
                rdma.start()
            return c
        lax.fori_loop(lo, hi, f, 0)

    for b in range(B):
        xb = x_ref[b]
        q_all = jnp.dot(xb, wq_ref[:, :],
                        preferred_element_type=jnp.float32)
        q_all = q_all.astype(jnp.bfloat16)
        if b == 0:
            for c in kv_copies:
                c.wait()
        acc = jnp.zeros((SQ, DM), jnp.float32)
        for h in range(H_PER):
            q = q_all[:, h * DH:(h + 1) * DH]
            k = k_vmem[b, :, h, :].astype(jnp.bfloat16)
            v = v_vmem[b, :, h, :].astype(jnp.bfloat16)
            s = lax.dot_general(
                q, k,
                (((1,), (1,)), ((), ())),
                preferred_element_type=jnp.float32,
            ) * 0.125
            s = jnp.where(mask, s, -1e9)
            m = jnp.max(s, axis=-1, keepdims=True)
            w = jnp.exp(s - m)
            w = w / jnp.sum(w, axis=-1, keepdims=True)
            ctx = jnp.dot(w.astype(jnp.bfloat16), v,
                          preferred_element_type=jnp.float32)
            acc = acc + jnp.dot(ctx.astype(jnp.bfloat16),
                                wo_ref[h * DH:(h + 1) * DH, :],
                                preferred_element_type=jnp.float32)
        acc_ref[b * SQ:(b + 1) * SQ, :] = acc.astype(jnp.bfloat16)
        if b == 0:
            pl.semaphore_wait(barrier, N_DEV - 1)
        _send_chunks(b * SQ // CHUNK, (b + 1) * SQ // CHUNK)

    gather_ref[me] = acc_ref[pl.ds(me * CHUNK, CHUNK), :]

    def _wait_a(o, c):
        src = jnp.mod(me + o, N_DEV)
        rdma = pltpu.make_async_remote_copy(
            src_ref=acc_ref.at[pl.ds(0, CHUNK), :],
            dst_ref=gather_ref.at[src],
            send_sem=send_a.at[src],
            recv_sem=recv_a.at[src],
            device_id=(src,), device_id_type=_MESH,
        )
        rdma.wait_recv()
        return c

    lax.fori_loop(1, N_DEV, _wait_a, 0)

    red = jnp.sum(gather_ref[:, :, :].astype(jnp.float32), axis=0)
    res_ref[pl.ds(me * CHUNK, CHUNK), :] = red.astype(jnp.bfloat16)

    def _send_b(o, c):
        tgt = jnp.mod(me + o, N_DEV)
        rdma = pltpu.make_async_remote_copy(
            src_ref=res_ref.at[pl.ds(me * CHUNK, CHUNK), :],
            dst_ref=res_ref.at[pl.ds(me * CHUNK, CHUNK), :],
            send_sem=send_b.at[tgt],
            recv_sem=recv_b.at[me],
            device_id=(tgt,), device_id_type=_MESH,
        )
        rdma.start()
        return c

    lax.fori_loop(1, N_DEV, _send_b, 0)

    def _wait_b(o, c):
        src = jnp.mod(me + o, N_DEV)
        rdma = pltpu.make_async_remote_copy(
            src_ref=res_ref.at[pl.ds(0, CHUNK), :],
            dst_ref=res_ref.at[pl.ds(src * CHUNK, CHUNK), :],
            send_sem=send_b.at[src],
            recv_sem=recv_b.at[src],
            device_id=(src,), device_id_type=_MESH,
        )
        rdma.wait_recv()
        return c

    lax.fori_loop(1, N_DEV, _wait_b, 0)

    def _drain(o, c):
        tgt = jnp.mod(me + o, N_DEV)
        for sem_arr in (send_a, send_b):
            rdma = pltpu.make_async_remote_copy(
                src_ref=acc_ref.at[pl.ds(0, CHUNK), :],
                dst_ref=gather_ref.at[0],
                send_sem=sem_arr.at[tgt],
                recv_sem=recv_a.at[0],
                device_id=(tgt,), device_id_type=_MESH,
            )
            rdma.wait_send()
        return c

    lax.fori_loop(1, N_DEV, _drain, 0)

    for b in range(B):
        out_ref[b, :, :] = res_ref[b * SQ:(b + 1) * SQ, :].astype(jnp.float32)


def kernel(x, Wq, K_ext, V_ext, Wo):
    xb = x.astype(jnp.bfloat16)
    wq = Wq.astype(jnp.bfloat16)
    wo = Wo.astype(jnp.bfloat16)

    return pl.pallas_call(
        _body,
        out_shape=jax.ShapeDtypeStruct((B, SQ, DM), jnp.float32),
        in_specs=[
            pl.BlockSpec(memory_space=pltpu.VMEM),
            pl.BlockSpec(memory_space=pltpu.VMEM),
            pl.BlockSpec(memory_space=pl.ANY),
            pl.BlockSpec(memory_space=pl.ANY),
            pl.BlockSpec(memory_space=pltpu.VMEM),
        ],
        out_specs=pl.BlockSpec(memory_space=pltpu.VMEM),
        scratch_shapes=[
            pltpu.VMEM((ROWS, DM), jnp.bfloat16),
            pltpu.VMEM((N_DEV, CHUNK, DM), jnp.bfloat16),
            pltpu.VMEM((ROWS, DM), jnp.bfloat16),
            pltpu.VMEM((B, SKV, H_PER, DH), jnp.float32),
            pltpu.VMEM((B, SKV, H_PER, DH), jnp.float32),
            pltpu.SemaphoreType.DMA((N_DEV,)),
            pltpu.SemaphoreType.DMA((N_DEV,)),
            pltpu.SemaphoreType.DMA((N_DEV,)),
            pltpu.SemaphoreType.DMA((N_DEV,)),
            pltpu.SemaphoreType.DMA((2,)),
        ],
        compiler_params=pltpu.CompilerParams(collective_id=0),
    )(xb, wq, K_ext, V_ext, wo)


# device time: 144874 ns/iter; 1.8318x vs baseline; 1.8318x over previous
import jax
import jax.numpy as jnp
from jax import lax
from jax.experimental import pallas as pl
from jax.experimental.pallas import tpu as pltpu

N_DEV = 32
B, SQ, SKV, DM = 2, 512, 512, 768
H_PER = 8
DH = 64
ROWS = B * SQ
CHUNK = ROWS // N_DEV

_MESH = pl.DeviceIdType.MESH


def _body(x_ref, wq_ref, k_ref, v_ref, wo_ref, out_ref,
          acc_ref, gather_ref, res_ref,
          send_a, recv_a, send_b, recv_b):
    me = lax.axis_index("i")

    barrier = pltpu.get_barrier_semaphore()

    def _sig(o, c):
        pl.semaphore_signal(barrier, inc=1,
                            device_id=(jnp.mod(me + o, N_DEV),),
                            device_id_type=_MESH)
        return c

    lax.fori_loop(1, N_DEV, _sig, 0)

    qi = lax.broadcasted_iota(jnp.int32, (SQ, SKV), 0)
    ki = lax.broadcasted_iota(jnp.int32, (SQ, SKV), 1)
    mask = (jnp.abs(qi - ki) <= 128) | (ki < 32) | (qi < 32)

    def _send_chunks(lo, hi):
        def f(t, c):
            @pl.when(t != me)
            def _():
                rdma = pltpu.make_async_remote_copy(
                    src_ref=acc_ref.at[pl.ds(t * CHUNK, CHUNK), :],
                    dst_ref=gather_ref.at[me],
                    send_sem=send_a.at[t],
                    recv_sem=recv_a.at[me],
                    device_id=(t,), device_id_type=_MESH,
                )
                rdma.start()
            return c
        lax.fori_loop(lo, hi, f, 0)

    for b in range(B):
        xb = x_ref[b]
        q_all = jnp.dot(xb, wq_ref[:, :],
                        preferred_element_type=jnp.float32)
        q_all = (q_all * 0.125).astype(jnp.bfloat16)
        acc = jnp.zeros((SQ, DM), jnp.float32)
        for h in range(H_PER):
            q = q_all[:, h * DH:(h + 1) * DH]
            k = k_ref[b, :, h, :]
            v = v_ref[b, :, h, :]
            s = lax.dot_general(
                q, k,
                (((1,), (1,)), ((), ())),
                preferred_element_type=jnp.float32,
            )
            w = jnp.exp(jnp.where(mask, s, -1e9).astype(jnp.bfloat16))
            wsum = jnp.sum(w.astype(jnp.float32), axis=-1, keepdims=True)
            ctx = jnp.dot(w, v,
                          preferred_element_type=jnp.float32)
            ctx = ctx / wsum
            acc = acc + jnp.dot(ctx.astype(jnp.bfloat16),
                                wo_ref[h * DH:(h + 1) * DH, :],
                                preferred_element_type=jnp.float32)
        acc_ref[b * SQ:(b + 1) * SQ, :] = acc.astype(jnp.bfloat16)
        if b == 0:
            pl.semaphore_wait(barrier, N_DEV - 1)
        _send_chunks(b * SQ // CHUNK, (b + 1) * SQ // CHUNK)

    gather_ref[me] = acc_ref[pl.ds(me * CHUNK, CHUNK), :]

    def _wait_a(o, c):
        src = jnp.mod(me + o, N_DEV)
        rdma = pltpu.make_async_remote_copy(
            src_ref=acc_ref.at[pl.ds(0, CHUNK), :],
            dst_ref=gather_ref.at[src],
            send_sem=send_a.at[src],
            recv_sem=recv_a.at[src],
            device_id=(src,), device_id_type=_MESH,
        )
        rdma.wait_recv()
        return c

    lax.fori_loop(1, N_DEV, _wait_a, 0)

    red = jnp.sum(gather_ref[:, :, :].astype(jnp.float32), axis=0)
    res_ref[pl.ds(me * CHUNK, CHUNK), :] = red.astype(jnp.bfloat16)

    def _send_b(o, c):
        tgt = jnp.mod(me + o, N_DEV)
        rdma = pltpu.make_async_remote_copy(
            src_ref=res_ref.at[pl.ds(me * CHUNK, CHUNK), :],
            dst_ref=res_ref.at[pl.ds(me * CHUNK, CHUNK), :],
            send_sem=send_b.at[tgt],
            recv_sem=recv_b.at[me],
            device_id=(tgt,), device_id_type=_MESH,
        )
        rdma.start()
        return c

    lax.fori_loop(1, N_DEV, _send_b, 0)

    def _wait_b(o, c):
        src = jnp.mod(me + o, N_DEV)
        rdma = pltpu.make_async_remote_copy(
            src_ref=res_ref.at[pl.ds(0, CHUNK), :],
            dst_ref=res_ref.at[pl.ds(src * CHUNK, CHUNK), :],
            send_sem=send_b.at[src],
            recv_sem=recv_b.at[src],
            device_id=(src,), device_id_type=_MESH,
        )
        rdma.wait_recv()
        return c

    lax.fori_loop(1, N_DEV, _wait_b, 0)

    def _drain(o, c):
        tgt = jnp.mod(me + o, N_DEV)
        for sem_arr in (send_a, send_b):
            rdma = pltpu.make_async_remote_copy(
                src_ref=acc_ref.at[pl.ds(0, CHUNK), :],
                dst_ref=gather_ref.at[0],
                send_sem=sem_arr.at[tgt],
                recv_sem=recv_a.at[0],
                device_id=(tgt,), device_id_type=_MESH,
            )
            rdma.wait_send()
        return c

    lax.fori_loop(1, N_DEV, _drain, 0)

    for b in range(B):
        out_ref[b, :, :] = res_ref[b * SQ:(b + 1) * SQ, :].astype(jnp.float32)


def kernel(x, Wq, K_ext, V_ext, Wo):
    xb = x.astype(jnp.bfloat16)
    wq = Wq.astype(jnp.bfloat16)
    wo = Wo.astype(jnp.bfloat16)
    me = lax.axis_index("i")
    k = lax.dynamic_slice_in_dim(K_ext, me * H_PER, H_PER, axis=2)
    v = lax.dynamic_slice_in_dim(V_ext, me * H_PER, H_PER, axis=2)
    k = k.astype(jnp.bfloat16)
    v = v.astype(jnp.bfloat16)

    return pl.pallas_call(
        _body,
        out_shape=jax.ShapeDtypeStruct((B, SQ, DM), jnp.float32),
        in_specs=[pl.BlockSpec(memory_space=pltpu.VMEM)] * 5,
        out_specs=pl.BlockSpec(memory_space=pltpu.VMEM),
        scratch_shapes=[
            pltpu.VMEM((ROWS, DM), jnp.bfloat16),
            pltpu.VMEM((N_DEV, CHUNK, DM), jnp.bfloat16),
            pltpu.VMEM((ROWS, DM), jnp.bfloat16),
            pltpu.SemaphoreType.DMA((N_DEV,)),
            pltpu.SemaphoreType.DMA((N_DEV,)),
            pltpu.SemaphoreType.DMA((N_DEV,)),
            pltpu.SemaphoreType.DMA((N_DEV,)),
        ],
        compiler_params=pltpu.CompilerParams(collective_id=0),
    )(xb, wq, k, v, wo)
